# baseline (speedup 1.0000x reference)
import functools

import jax
import jax.numpy as jnp
from jax import lax
from jax.experimental import pallas as pl
from jax.experimental.pallas import tpu as pltpu
from jax.experimental.pallas import tpu_sc as plsc

_VOCAB = 1000000
_D = 64
_B = 16384
_NC = 2
_NS = 16
_NW = _NC * _NS
_BPW = _B // _NW
_HB = 128


def _gather_body(word_hbm, ctx_hbm, emb_hbm, ctxtab_hbm, u_hbm, v_hbm,
                 idx_u, idx_v, sh_u, sh_v, rows_u, rows_v,
                 sem_u, sem_v, sem_i):
    wid = lax.axis_index("s") * _NC + lax.axis_index("c")
    sid = lax.axis_index("s")
    base = wid * _BPW
    pltpu.async_copy(word_hbm.at[pl.ds(base, _BPW)], idx_u, sem_i).wait()
    pltpu.async_copy(ctx_hbm.at[pl.ds(base, _BPW)], idx_v, sem_i).wait()

    for h in range(_BPW // _HB):
        hb = h * _HB

        @pl.loop(0, _HB, step=16)
        def _(r):
            iu = idx_u[pl.ds(hb + r, 16)]
            iv = idx_v[pl.ds(hb + r, 16)]
            for j in range(16):
                pltpu.async_copy(emb_hbm.at[pl.ds(iu[j], 1)],
                                 sh_u.at[sid, pl.ds(r + j, 1)], sem_u)
                pltpu.async_copy(ctxtab_hbm.at[pl.ds(iv[j], 1)],
                                 sh_v.at[sid, pl.ds(r + j, 1)], sem_v)

        pltpu.make_async_copy(emb_hbm.at[pl.ds(0, _HB)], sh_u.at[sid],
                              sem_u).wait()
        pltpu.make_async_copy(ctxtab_hbm.at[pl.ds(0, _HB)], sh_v.at[sid],
                              sem_v).wait()

        pltpu.sync_copy(sh_u.at[sid], rows_u)
        pltpu.sync_copy(sh_v.at[sid], rows_v)
        pltpu.sync_copy(rows_u, u_hbm.at[pl.ds(base + hb, _HB)])
        pltpu.sync_copy(rows_v, v_hbm.at[pl.ds(base + hb, _HB)])


def _loss_body(u_ref, v_ref, loss_ref):
    p = u_ref[...] * v_ref[...]
    s = jnp.sum(p, axis=1)
    ls = jnp.minimum(s, 0.0) - jnp.log1p(jnp.exp(-jnp.abs(s)))
    loss_ref[0, 0] = -jnp.sum(ls) * (1.0 / _B)


@jax.jit
def kernel(word, context, emb_table, ctx_table):
    mesh = plsc.VectorSubcoreMesh(core_axis_name="c", subcore_axis_name="s")
    gather = pl.kernel(
        _gather_body,
        out_type=[jax.ShapeDtypeStruct((_B, _D), jnp.float32),
                  jax.ShapeDtypeStruct((_B, _D), jnp.float32)],
        mesh=mesh,
        scratch_types=[
            pltpu.VMEM((_BPW,), jnp.int32),
            pltpu.VMEM((_BPW,), jnp.int32),
            pltpu.VMEM_SHARED((_NS, _HB, _D), jnp.float32),
            pltpu.VMEM_SHARED((_NS, _HB, _D), jnp.float32),
            pltpu.VMEM((_HB, _D), jnp.float32),
            pltpu.VMEM((_HB, _D), jnp.float32),
            pltpu.SemaphoreType.DMA,
            pltpu.SemaphoreType.DMA,
            pltpu.SemaphoreType.DMA,
        ],
        compiler_params=pltpu.CompilerParams(use_tc_tiling_on_sc=True),
    )
    embed_u, embed_v = gather(word, context, emb_table, ctx_table)
    loss2 = pl.pallas_call(
        _loss_body,
        out_shape=jax.ShapeDtypeStruct((1, 1), jnp.float32),
        out_specs=pl.BlockSpec(memory_space=pltpu.SMEM),
    )(embed_u, embed_v)
    return loss2[0, 0], embed_u
